# baseline (speedup 1.0000x reference)
import functools

import jax
import jax.numpy as jnp
from jax import lax
from jax.experimental import pallas as pl
from jax.experimental.pallas import tpu as pltpu
from jax.experimental.pallas import tpu_sc as plsc

B, S, D = 4, 4096, 4096
SD = S * D
NC, NS = 2, 16
NW = NC * NS
WPS = SD // NW
C = 8192
NCH = WPS // C

_mesh = plsc.VectorSubcoreMesh(core_axis_name="c", subcore_axis_name="s")


@functools.partial(
    pl.kernel,
    out_type=jax.ShapeDtypeStruct((B * SD,), jnp.float32),
    mesh=_mesh,
    scratch_types=[
        pltpu.VMEM((B, C), jnp.float32),
        pltpu.VMEM((C,), jnp.float32),
    ],
)
def _sc_add(x_hbm, pe_hbm, out_hbm, xb, pb):
    wid = lax.axis_index("s") * NC + lax.axis_index("c")
    ps = wid * WPS

    def chunk(g, carry):
        po = ps + g * C
        pltpu.sync_copy(pe_hbm.at[pl.ds(po, C)], pb)
        for b in range(B):
            xo = b * SD + po
            pltpu.sync_copy(x_hbm.at[pl.ds(xo, C)], xb.at[b])

            def add16(i, c, b=b):
                sl = pl.ds(i * 16, 16)
                plsc.addupdate(xb.at[b].at[sl], pb[sl])
                return c

            lax.fori_loop(0, C // 16, add16, None)
            pltpu.sync_copy(xb.at[b], out_hbm.at[pl.ds(xo, C)])
        return carry

    lax.fori_loop(0, NCH, chunk, None)


def kernel(x, pos_emb_weight):
    out = _sc_add(x.reshape(-1), pos_emb_weight.reshape(-1))
    return out.reshape(x.shape)


# speedup vs baseline: 1.2844x; 1.2844x over previous
import functools

import jax
import jax.numpy as jnp
from jax import lax
from jax.experimental import pallas as pl
from jax.experimental.pallas import tpu as pltpu
from jax.experimental.pallas import tpu_sc as plsc

B, S, D = 4, 4096, 4096
SD = S * D
NC, NS = 2, 16
NW = NC * NS
WPS = SD // NW
RING = 4
C = 4096
NCH = WPS // C

_mesh = plsc.VectorSubcoreMesh(core_axis_name="c", subcore_axis_name="s")


@functools.partial(
    pl.kernel,
    out_type=jax.ShapeDtypeStruct((B * SD,), jnp.float32),
    mesh=_mesh,
    scratch_types=[
        pltpu.VMEM((RING, B, C), jnp.float32),
        pltpu.VMEM((RING, C), jnp.float32),
        pltpu.SemaphoreType.DMA((RING,)),
        pltpu.SemaphoreType.DMA((RING,)),
        pltpu.SemaphoreType.DMA((RING,)),
    ],
)
def _sc_add(x_hbm, pe_hbm, out_hbm, xb, pb, sx, sp, so):
    wid = lax.axis_index("s") * NC + lax.axis_index("c")
    ps = wid * WPS

    def fire_in(g, s):
        po = ps + g * C
        pltpu.async_copy(pe_hbm.at[pl.ds(po, C)], pb.at[s], sp.at[s])
        for b in range(B):
            pltpu.async_copy(x_hbm.at[pl.ds(b * SD + po, C)], xb.at[s, b], sx.at[s])

    def wait_in(g, s):
        po = ps + g * C
        pltpu.make_async_copy(pe_hbm.at[pl.ds(po, C)], pb.at[s], sp.at[s]).wait()
        for b in range(B):
            pltpu.make_async_copy(
                x_hbm.at[pl.ds(b * SD + po, C)], xb.at[s, b], sx.at[s]
            ).wait()

    def fire_out(g, s):
        po = ps + g * C
        for b in range(B):
            pltpu.async_copy(xb.at[s, b], out_hbm.at[pl.ds(b * SD + po, C)], so.at[s])

    def wait_out(g, s):
        po = ps + g * C
        for b in range(B):
            pltpu.make_async_copy(
                xb.at[s, b], out_hbm.at[pl.ds(b * SD + po, C)], so.at[s]
            ).wait()

    fire_in(0, 0)
    fire_in(1, 1)

    def body(g, carry):
        s = g & (RING - 1)
        wait_in(g, s)
        for b in range(B):

            def add16(i, c, b=b):
                sl = pl.ds(i * 16, 16)
                plsc.addupdate(xb.at[s].at[b].at[sl], pb[s, sl])
                return c

            lax.fori_loop(0, C // 16, add16, None)
        fire_out(g, s)
        sn = (g + 2) & (RING - 1)

        @pl.when(g >= 2)
        def _():
            wait_out(g - 2, sn)

        fire_in(g + 2, sn)
        return carry

    lax.fori_loop(0, NCH, body, None)

    for g in (NCH - 2, NCH - 1):
        wait_out(g, g & (RING - 1))
    for g in (NCH, NCH + 1):
        wait_in(g, g & (RING - 1))


def kernel(x, pos_emb_weight):
    out = _sc_add(x.reshape(-1), pos_emb_weight.reshape(-1))
    return out.reshape(x.shape)


# speedup vs baseline: 1.7010x; 1.3243x over previous
import functools

import jax
import jax.numpy as jnp
from jax import lax
from jax.experimental import pallas as pl
from jax.experimental.pallas import tpu as pltpu
from jax.experimental.pallas import tpu_sc as plsc

B, S, D = 4, 4096, 4096
SD = S * D
NC, NS = 2, 16
NW = NC * NS
WPS = SD // NW
RING = 4
C = 4096
NCH = WPS // C

_mesh = plsc.VectorSubcoreMesh(core_axis_name="c", subcore_axis_name="s")


@functools.partial(
    pl.kernel,
    out_type=jax.ShapeDtypeStruct((B * SD,), jnp.float32),
    mesh=_mesh,
    scratch_types=[
        pltpu.VMEM((RING, B, C), jnp.float32),
        pltpu.VMEM((RING, C), jnp.float32),
        pltpu.SemaphoreType.DMA((RING,)),
        pltpu.SemaphoreType.DMA((RING,)),
        pltpu.SemaphoreType.DMA((RING,)),
    ],
)
def _sc_add(x_hbm, pe_hbm, out_hbm, xb, pb, sx, sp, so):
    wid = lax.axis_index("s") * NC + lax.axis_index("c")
    ps = wid * WPS

    def fire_in(g, s):
        po = ps + g * C
        pltpu.async_copy(pe_hbm.at[pl.ds(po, C)], pb.at[s], sp.at[s])
        for b in range(B):
            pltpu.async_copy(x_hbm.at[pl.ds(b * SD + po, C)], xb.at[s, b], sx.at[s])

    def wait_in(g, s):
        po = ps + g * C
        pltpu.make_async_copy(pe_hbm.at[pl.ds(po, C)], pb.at[s], sp.at[s]).wait()
        for b in range(B):
            pltpu.make_async_copy(
                x_hbm.at[pl.ds(b * SD + po, C)], xb.at[s, b], sx.at[s]
            ).wait()

    def fire_out(g, s):
        po = ps + g * C
        for b in range(B):
            pltpu.async_copy(xb.at[s, b], out_hbm.at[pl.ds(b * SD + po, C)], so.at[s])

    def wait_out(g, s):
        po = ps + g * C
        for b in range(B):
            pltpu.make_async_copy(
                xb.at[s, b], out_hbm.at[pl.ds(b * SD + po, C)], so.at[s]
            ).wait()

    fire_in(0, 0)
    fire_in(1, 1)

    def body(g, carry):
        s = g & (RING - 1)
        wait_in(g, s)
        for b in range(B):

            @plsc.parallel_loop(0, C, step=16, unroll=8)
            def add16(i, b=b):
                sl = pl.ds(i, 16)
                plsc.addupdate(xb.at[s].at[b].at[sl], pb[s, sl])
        fire_out(g, s)
        sn = (g + 2) & (RING - 1)

        @pl.when(g >= 2)
        def _():
            wait_out(g - 2, sn)

        fire_in(g + 2, sn)
        return carry

    lax.fori_loop(0, NCH, body, None)

    for g in (NCH - 2, NCH - 1):
        wait_out(g, g & (RING - 1))
    for g in (NCH, NCH + 1):
        wait_in(g, g & (RING - 1))


def kernel(x, pos_emb_weight):
    out = _sc_add(x.reshape(-1), pos_emb_weight.reshape(-1))
    return out.reshape(x.shape)


# speedup vs baseline: 6.3951x; 3.7596x over previous
import functools

import jax
import jax.numpy as jnp
from jax import lax
from jax.experimental import pallas as pl
from jax.experimental.pallas import tpu as pltpu
from jax.experimental.pallas import tpu_sc as plsc

B, S, D = 4, 4096, 4096
T = 8192
NC, NS = 2, 16
NW = NC * NS
CR, CC = 8, 512
DG = D // CC
NCHT = (S // CR) * DG
NCH = NCHT // NW
RING = 4

_mesh = plsc.VectorSubcoreMesh(core_axis_name="c", subcore_axis_name="s")


@functools.partial(
    pl.kernel,
    out_type=jax.ShapeDtypeStruct((B, S, D), jnp.float32),
    mesh=_mesh,
    scratch_types=[
        pltpu.VMEM((RING, B, CR, CC), jnp.float32),
        pltpu.VMEM((RING, CR, CC), jnp.float32),
        pltpu.SemaphoreType.DMA((RING,)),
        pltpu.SemaphoreType.DMA((RING,)),
        pltpu.SemaphoreType.DMA((RING,)),
    ],
    compiler_params=pltpu.CompilerParams(use_tc_tiling_on_sc=True),
)
def _sc_add(x_hbm, pe_hbm, out_hbm, xb, pb, sx, sp, so):
    wid = lax.axis_index("s") * NC + lax.axis_index("c")
    c0 = wid * NCH

    def _chunk(g):
        c = (c0 + g) & (NCHT - 1)
        s0 = (c // DG) * CR
        d0 = (c % DG) * CC
        return s0, d0

    def fire_in(g, s):
        s0, d0 = _chunk(g)
        pltpu.async_copy(
            pe_hbm.at[pl.ds(s0, CR), pl.ds(d0, CC)], pb.at[s], sp.at[s]
        )
        for b in range(B):
            pltpu.async_copy(
                x_hbm.at[b, pl.ds(s0, CR), pl.ds(d0, CC)], xb.at[s, b], sx.at[s]
            )

    def wait_in(g, s):
        s0, d0 = _chunk(g)
        pltpu.make_async_copy(
            pe_hbm.at[pl.ds(s0, CR), pl.ds(d0, CC)], pb.at[s], sp.at[s]
        ).wait()
        for b in range(B):
            pltpu.make_async_copy(
                x_hbm.at[b, pl.ds(s0, CR), pl.ds(d0, CC)], xb.at[s, b], sx.at[s]
            ).wait()

    def fire_out(g, s):
        s0, d0 = _chunk(g)
        for b in range(B):
            pltpu.async_copy(
                xb.at[s, b], out_hbm.at[b, pl.ds(s0, CR), pl.ds(d0, CC)], so.at[s]
            )

    def wait_out(g, s):
        s0, d0 = _chunk(g)
        for b in range(B):
            pltpu.make_async_copy(
                xb.at[s, b], out_hbm.at[b, pl.ds(s0, CR), pl.ds(d0, CC)], so.at[s]
            ).wait()

    fire_in(0, 0)
    fire_in(1, 1)

    def body(g, carry):
        s = g & (RING - 1)
        wait_in(g, s)
        for b in range(B):
            for r in range(CR):

                @plsc.parallel_loop(0, CC, step=16, unroll=8)
                def add16(i, b=b, r=r):
                    sl = pl.ds(i, 16)
                    plsc.addupdate(xb.at[s].at[b].at[r].at[sl], pb[s, r, sl])

        fire_out(g, s)
        sn = (g + 2) & (RING - 1)

        @pl.when(g >= 2)
        def _():
            wait_out(g - 2, sn)

        fire_in(g + 2, sn)
        return carry

    lax.fori_loop(0, NCH, body, None)

    for g in (NCH - 2, NCH - 1):
        wait_out(g, g & (RING - 1))
    for g in (NCH, NCH + 1):
        wait_in(g, g & (RING - 1))


def kernel(x, pos_emb_weight):
    return _sc_add(x, pos_emb_weight)


# speedup vs baseline: 6.6169x; 1.0347x over previous
import functools

import jax
import jax.numpy as jnp
from jax import lax
from jax.experimental import pallas as pl
from jax.experimental.pallas import tpu as pltpu
from jax.experimental.pallas import tpu_sc as plsc

B, S, D = 4, 4096, 4096
T = 8192
NC, NS = 2, 16
NW = NC * NS
CR, CC = 8, 512
DG = D // CC
NCHT = (S // CR) * DG
NCH = NCHT // NW
RING = 4

_mesh = plsc.VectorSubcoreMesh(core_axis_name="c", subcore_axis_name="s")


@functools.partial(
    pl.kernel,
    out_type=jax.ShapeDtypeStruct((B, S, D), jnp.float32),
    mesh=_mesh,
    scratch_types=[
        pltpu.VMEM((RING, B, CR, CC), jnp.float32),
        pltpu.VMEM((RING, CR, CC), jnp.float32),
        pltpu.SemaphoreType.DMA((RING,)),
        pltpu.SemaphoreType.DMA((RING,)),
        pltpu.SemaphoreType.DMA((RING,)),
    ],
    compiler_params=pltpu.CompilerParams(use_tc_tiling_on_sc=True),
)
def _sc_add(x_hbm, pe_hbm, out_hbm, xb, pb, sx, sp, so):
    wid = lax.axis_index("s") * NC + lax.axis_index("c")
    c0 = wid * NCH

    def _chunk(g):
        c = (c0 + g) & (NCHT - 1)
        s0 = (c // DG) * CR
        d0 = (c % DG) * CC
        return s0, d0

    def fire_in(g, s):
        s0, d0 = _chunk(g)
        pltpu.async_copy(
            pe_hbm.at[pl.ds(s0, CR), pl.ds(d0, CC)], pb.at[s], sp.at[s]
        )
        for b in range(B):
            pltpu.async_copy(
                x_hbm.at[b, pl.ds(s0, CR), pl.ds(d0, CC)], xb.at[s, b], sx.at[s]
            )

    def wait_in(g, s):
        s0, d0 = _chunk(g)
        pltpu.make_async_copy(
            pe_hbm.at[pl.ds(s0, CR), pl.ds(d0, CC)], pb.at[s], sp.at[s]
        ).wait()
        for b in range(B):
            pltpu.make_async_copy(
                x_hbm.at[b, pl.ds(s0, CR), pl.ds(d0, CC)], xb.at[s, b], sx.at[s]
            ).wait()

    def fire_out(g, s):
        s0, d0 = _chunk(g)
        for b in range(B):
            pltpu.async_copy(
                xb.at[s, b], out_hbm.at[b, pl.ds(s0, CR), pl.ds(d0, CC)], so.at[s]
            )

    def wait_out(g, s):
        s0, d0 = _chunk(g)
        for b in range(B):
            pltpu.make_async_copy(
                xb.at[s, b], out_hbm.at[b, pl.ds(s0, CR), pl.ds(d0, CC)], so.at[s]
            ).wait()

    fire_in(0, 0)
    fire_in(1, 1)

    def body(g, carry):
        s = g & (RING - 1)
        wait_in(g, s)

        @plsc.parallel_loop(0, CR * CC, step=16, unroll=4)
        def add16(i):
            r = i >> 9
            col = pl.multiple_of(i & (CC - 1), 16)
            sl = pl.ds(col, 16)
            v = pb[s, r, sl]
            for b in range(B):
                plsc.addupdate(xb.at[s].at[b].at[r].at[sl], v)

        fire_out(g, s)
        sn = (g + 2) & (RING - 1)

        @pl.when(g >= 2)
        def _():
            wait_out(g - 2, sn)

        fire_in(g + 2, sn)
        return carry

    lax.fori_loop(0, NCH, body, None)

    for g in (NCH - 2, NCH - 1):
        wait_out(g, g & (RING - 1))
    for g in (NCH, NCH + 1):
        wait_in(g, g & (RING - 1))


def kernel(x, pos_emb_weight):
    return _sc_add(x, pos_emb_weight)


# speedup vs baseline: 6.7080x; 1.0138x over previous
import functools

import jax
import jax.numpy as jnp
from jax import lax
from jax.experimental import pallas as pl
from jax.experimental.pallas import tpu as pltpu
from jax.experimental.pallas import tpu_sc as plsc

B, S, D = 4, 4096, 4096
T = 8192
NC, NS = 2, 16
NW = NC * NS
CR, CC = 8, 1024
DG = D // CC
NCHT = (S // CR) * DG
NCH = NCHT // NW
RING = 3

_mesh = plsc.VectorSubcoreMesh(core_axis_name="c", subcore_axis_name="s")


@functools.partial(
    pl.kernel,
    out_type=jax.ShapeDtypeStruct((B, S, D), jnp.float32),
    mesh=_mesh,
    scratch_types=[
        pltpu.VMEM((RING, B, CR, CC), jnp.float32),
        pltpu.VMEM((RING, CR, CC), jnp.float32),
        pltpu.SemaphoreType.DMA((RING,)),
        pltpu.SemaphoreType.DMA((RING,)),
        pltpu.SemaphoreType.DMA((RING,)),
    ],
    compiler_params=pltpu.CompilerParams(use_tc_tiling_on_sc=True),
)
def _sc_add(x_hbm, pe_hbm, out_hbm, xb, pb, sx, sp, so):
    wid = lax.axis_index("s") * NC + lax.axis_index("c")
    c0 = wid * NCH

    def _chunk(g):
        c = (c0 + g) & (NCHT - 1)
        s0 = (c // DG) * CR
        d0 = (c % DG) * CC
        return s0, d0

    def fire_in(g, s):
        s0, d0 = _chunk(g)
        pltpu.async_copy(
            pe_hbm.at[pl.ds(s0, CR), pl.ds(d0, CC)], pb.at[s], sp.at[s]
        )
        for b in range(B):
            pltpu.async_copy(
                x_hbm.at[b, pl.ds(s0, CR), pl.ds(d0, CC)], xb.at[s, b], sx.at[s]
            )

    def wait_in(g, s):
        s0, d0 = _chunk(g)
        pltpu.make_async_copy(
            pe_hbm.at[pl.ds(s0, CR), pl.ds(d0, CC)], pb.at[s], sp.at[s]
        ).wait()
        for b in range(B):
            pltpu.make_async_copy(
                x_hbm.at[b, pl.ds(s0, CR), pl.ds(d0, CC)], xb.at[s, b], sx.at[s]
            ).wait()

    def fire_out(g, s):
        s0, d0 = _chunk(g)
        for b in range(B):
            pltpu.async_copy(
                xb.at[s, b], out_hbm.at[b, pl.ds(s0, CR), pl.ds(d0, CC)], so.at[s]
            )

    def wait_out(g, s):
        s0, d0 = _chunk(g)
        for b in range(B):
            pltpu.make_async_copy(
                xb.at[s, b], out_hbm.at[b, pl.ds(s0, CR), pl.ds(d0, CC)], so.at[s]
            ).wait()

    fire_in(0, 0)
    fire_in(1, 1)

    def body(g, carry):
        s = lax.rem(g, RING)
        wait_in(g, s)

        @plsc.parallel_loop(0, CR * CC, step=16, unroll=4)
        def add16(i):
            r = i >> 10
            col = pl.multiple_of(i & (CC - 1), 16)
            sl = pl.ds(col, 16)
            v = pb[s, r, sl]
            for b in range(B):
                plsc.addupdate(xb.at[s].at[b].at[r].at[sl], v)

        fire_out(g, s)
        sn = lax.rem(g + 2, RING)

        @pl.when(g >= 1)
        def _():
            wait_out(g - 1, sn)

        fire_in(g + 2, sn)
        return carry

    lax.fori_loop(0, NCH, body, None)

    wait_out(NCH - 1, (NCH - 1) % RING)
    for g in (NCH, NCH + 1):
        wait_in(g, g % RING)


def kernel(x, pos_emb_weight):
    return _sc_add(x, pos_emb_weight)
